# baseline (speedup 1.0000x reference)
import dataclasses
import functools

import jax
import jax.numpy as jnp
from jax import lax
from jax.experimental import pallas as pl
from jax.experimental.pallas import tpu as pltpu
from jax.experimental.pallas import tpu_sc as plsc

_LANES = 16
_CHUNK = 80
_N_TILES = 32
_N_SLOTS = 3


def _transform(h, W):
    n, d_in = h.shape
    r, _, d_out = W.shape

    def mm_kernel(h_ref, w_ref, out_ref):
        out_ref[0] = jnp.dot(h_ref[...], w_ref[0],
                             preferred_element_type=jnp.float32)

    return pl.pallas_call(
        mm_kernel,
        grid=(r,),
        in_specs=[
            pl.BlockSpec((n, d_in), lambda i: (0, 0)),
            pl.BlockSpec((1, d_in, d_out), lambda i: (i, 0, 0)),
        ],
        out_specs=pl.BlockSpec((1, n, d_out), lambda i: (i, 0, 0)),
        out_shape=jax.ShapeDtypeStruct((r, n, d_out), jnp.float32),
    )(h, W)


def _combine(partial):
    def add_kernel(p_ref, o_ref):
        o_ref[...] = p_ref[0] + p_ref[1]

    return pl.pallas_call(
        add_kernel,
        out_shape=jax.ShapeDtypeStruct(partial.shape[1:], jnp.float32),
    )(partial)


def _sc_edge_aggregate(t_flat, src, rel, dst, norm_flat, n_nodes):
    rn, d = t_flat.shape
    e = src.shape[0]
    c = _CHUNK
    assert e % (_N_TILES * c) == 0
    cpt = e // (_N_TILES * c)
    assert cpt >= 2 * _N_SLOTS
    assert n_nodes % 8 == 0
    npt = pl.cdiv(pl.cdiv(n_nodes, 16), c) * c
    acc_rows = 16 * npt
    full_tiles = n_nodes // npt
    tail_rows = n_nodes % npt
    assert tail_rows % 8 == 0
    nd16 = d // _LANES

    mesh = plsc.VectorSubcoreMesh(core_axis_name="c", subcore_axis_name="s")
    cp = pltpu.CompilerParams()
    if "needs_layout_passes" in pltpu.CompilerParams.__dataclass_fields__:
        cp = dataclasses.replace(cp, needs_layout_passes=False)

    slot_set = [
        pltpu.VMEM((4 * c,), jnp.int32),
        pltpu.VMEM((c,), jnp.int32),
        pltpu.VMEM((c, d), jnp.float32),
        pltpu.VMEM((c,), jnp.int32),
        pltpu.SemaphoreType.DMA,
        pltpu.SemaphoreType.DMA,
        pltpu.SemaphoreType.DMA,
    ]

    @functools.partial(
        pl.kernel,
        compiler_params=cp,
        out_type=jax.ShapeDtypeStruct((2 * n_nodes, d), jnp.float32),
        mesh=mesh,
        scratch_types=_N_SLOTS * slot_set + [
            pltpu.VMEM_SHARED((acc_rows, d), jnp.float32),
        ],
    )
    def sck(t_hbm, ed_hbm, out_hbm, *scratch):
        ns = len(slot_set)
        slots = tuple(scratch[i * ns:(i + 1) * ns] for i in range(_N_SLOTS))
        acc = scratch[_N_SLOTS * ns]
        core = lax.axis_index("c")
        sub = lax.axis_index("s")
        w = core * 16 + sub
        base_chunk = w * cpt
        zero16 = jnp.zeros((_LANES,), jnp.float32)

        def idx_copy(b, k):
            edb, _, _, _, semi, _, _ = slots[b]
            be = (base_chunk + k) * 4 * c
            return pltpu.make_async_copy(ed_hbm.at[pl.ds(be, 4 * c)], edb,
                                         semi)

        def issue_idx(b, k):
            idx_copy(b, k).start()

        def wait_idx(b, k):
            idx_copy(b, k).wait()

        def gidx_compute(b):
            edb, gidxb, _, _, _, _, _ = slots[b]
            for k16 in range(c // _LANES):
                off = k16 * _LANES
                sl = pl.ds(off, _LANES)
                gidxb[sl] = (edb[pl.ds(c + off, _LANES)] * n_nodes
                             + edb[pl.ds(off, _LANES)])

        def gather_copy(b):
            _, gidxb, rows, _, _, semg, _ = slots[b]
            return pltpu.make_async_copy(t_hbm.at[gidxb], rows, semg)

        def scale(b):
            edb, _, rows, dsts, _, _, _ = slots[b]

            @pl.loop(0, c, step=4)
            def _scale(i):
                for u in range(4):
                    ii = i + u
                    nb_i = plsc.load_gather(
                        edb, [jnp.full((_LANES,), 3 * c + ii, jnp.int32)])
                    nb = plsc.bitcast(nb_i, jnp.float32)
                    for kk in range(nd16):
                        sl = pl.ds(kk * _LANES, _LANES)
                        rows[ii, sl] = rows[ii, sl] * nb

            for k16 in range(c // _LANES):
                off = k16 * _LANES
                dsts[pl.ds(off, _LANES)] = edb[pl.ds(2 * c + off, _LANES)]

        def scatter_desc(b):
            _, _, rows, dsts, _, _, sema = slots[b]
            return pltpu.make_async_copy(rows, acc.at[dsts], sema)

        rows0 = slots[0][2]

        @pl.loop(0, c)
        def _zero_rows(i):
            for k in range(nd16):
                rows0[i, pl.ds(k * _LANES, _LANES)] = zero16

        row0 = sub * npt
        for jb in range(npt // c):
            pltpu.sync_copy(rows0, acc.at[pl.ds(row0 + jb * c, c)])

        plsc.subcore_barrier()

        for b in range(_N_SLOTS):
            issue_idx(b, b)
        for b in range(_N_SLOTS - 1):
            wait_idx(b, b)
            gidx_compute(b)
            gather_copy(b).start()

        def body(b, k, static_tail=False):
            b2 = (b + 2) % _N_SLOTS

            if not static_tail:
                @pl.when(k + 2 < cpt)
                def _prefetch_gather():
                    wait_idx(b2, k + 2)
                    gidx_compute(b2)

                    @pl.when(k >= 1)
                    def _drain_scatter():
                        scatter_desc(b2).wait()

                    gather_copy(b2).start()

            gather_copy(b).wait()
            scale(b)
            scatter_desc(b).start(add=True)

            if not static_tail:
                @pl.when(k + 3 < cpt)
                def _prefetch_idx():
                    issue_idx(b, k + 3)

        n_main = (cpt - 2) // _N_SLOTS

        @pl.loop(0, n_main)
        def _main(t):
            k = _N_SLOTS * t
            for b in range(_N_SLOTS):
                body(b, k + b)

        for k in range(_N_SLOTS * n_main, cpt):
            b = k % _N_SLOTS
            if k + 2 < cpt:
                b2 = (b + 2) % _N_SLOTS
                wait_idx(b2, k + 2)
                gidx_compute(b2)
                if k >= 1:
                    scatter_desc(b2).wait()
                gather_copy(b2).start()
            body(b, k, static_tail=True)

        for m in range(cpt - _N_SLOTS, cpt):
            scatter_desc(m % _N_SLOTS).wait()

        plsc.subcore_barrier()

        @pl.when(sub < full_tiles)
        def _write_full():
            pltpu.sync_copy(acc.at[pl.ds(row0, npt)],
                            out_hbm.at[pl.ds(core * n_nodes + row0, npt)])

        if tail_rows:
            @pl.when(sub == full_tiles)
            def _write_tail():
                pltpu.sync_copy(
                    acc.at[pl.ds(row0, tail_rows)],
                    out_hbm.at[pl.ds(core * n_nodes + row0, tail_rows)])

    norm_bits = jax.lax.bitcast_convert_type(norm_flat, jnp.int32)
    edata = jnp.stack([src, rel, dst, norm_bits], axis=0)
    edata = edata.reshape(4, e // c, c).transpose(1, 0, 2).reshape(-1)
    return sck(t_flat, edata)


def kernel(h, edge_index, rel_type, norm, W):
    n, d_in = h.shape
    r, _, d_out = W.shape
    e = rel_type.shape[0]
    transformed = _transform(h.astype(jnp.bfloat16),
                             W.astype(jnp.bfloat16)).reshape(r * n, d_out)
    src = edge_index[0]
    dst = edge_index[1]
    partial = _sc_edge_aggregate(transformed, src, rel_type, dst,
                                 norm.reshape(e), n)
    return _combine(partial.reshape(2, n, d_out))


# speedup vs baseline: 1.1789x; 1.1789x over previous
import dataclasses
import functools

import jax
import jax.numpy as jnp
from jax import lax
from jax.experimental import pallas as pl
from jax.experimental.pallas import tpu as pltpu
from jax.experimental.pallas import tpu_sc as plsc

_LANES = 16
_CHUNK = 80
_N_TILES = 32
_N_SLOTS = 3


def _transform(h, W):
    n, d_in = h.shape
    r, _, d_out = W.shape

    def mm_kernel(h_ref, w_ref, out_ref):
        out_ref[0] = jnp.dot(h_ref[...], w_ref[0],
                             preferred_element_type=jnp.float32)

    return pl.pallas_call(
        mm_kernel,
        grid=(r,),
        in_specs=[
            pl.BlockSpec((n, d_in), lambda i: (0, 0)),
            pl.BlockSpec((1, d_in, d_out), lambda i: (i, 0, 0)),
        ],
        out_specs=pl.BlockSpec((1, n, d_out), lambda i: (i, 0, 0)),
        out_shape=jax.ShapeDtypeStruct((r, n, d_out), jnp.float32),
    )(h, W)


def _combine(partial):
    def add_kernel(p_ref, o_ref):
        o_ref[...] = p_ref[0] + p_ref[1]

    return pl.pallas_call(
        add_kernel,
        out_shape=jax.ShapeDtypeStruct(partial.shape[1:], jnp.float32),
    )(partial)


def _sc_edge_aggregate(t_flat, src, rel, dst, norm_flat, n_nodes):
    rn, d = t_flat.shape
    e = src.shape[0]
    c = _CHUNK
    assert e % (_N_TILES * c) == 0
    cpt = e // (_N_TILES * c)
    assert cpt >= 2 * _N_SLOTS
    assert n_nodes % 8 == 0
    npt = pl.cdiv(pl.cdiv(n_nodes, 16), c) * c
    acc_rows = 16 * npt
    full_tiles = n_nodes // npt
    tail_rows = n_nodes % npt
    assert tail_rows % 8 == 0
    nd16 = d // _LANES

    mesh = plsc.VectorSubcoreMesh(core_axis_name="c", subcore_axis_name="s")
    cp = pltpu.CompilerParams()
    if "needs_layout_passes" in pltpu.CompilerParams.__dataclass_fields__:
        cp = dataclasses.replace(cp, needs_layout_passes=False)

    slot_set = [
        pltpu.VMEM((c,), jnp.int32),
        pltpu.VMEM((c,), jnp.int32),
        pltpu.VMEM((c,), jnp.int32),
        pltpu.VMEM((c,), jnp.int32),
        pltpu.VMEM((c,), jnp.float32),
        pltpu.VMEM((c, d), jnp.float32),
        pltpu.VMEM((c,), jnp.int32),
        pltpu.SemaphoreType.DMA,
        pltpu.SemaphoreType.DMA,
        pltpu.SemaphoreType.DMA,
    ]

    @functools.partial(
        pl.kernel,
        compiler_params=cp,
        out_type=jax.ShapeDtypeStruct((2 * n_nodes, d), jnp.float32),
        mesh=mesh,
        scratch_types=_N_SLOTS * slot_set + [
            pltpu.VMEM_SHARED((acc_rows, d), jnp.float32),
        ],
    )
    def sck(t_hbm, src_hbm, rel_hbm, dst_hbm, norm_hbm, out_hbm, *scratch):
        ns = len(slot_set)
        slots = tuple(scratch[i * ns:(i + 1) * ns] for i in range(_N_SLOTS))
        acc = scratch[_N_SLOTS * ns]
        core = lax.axis_index("c")
        sub = lax.axis_index("s")
        w = core * 16 + sub
        base_chunk = w * cpt
        zero16 = jnp.zeros((_LANES,), jnp.float32)

        def idx_copies(b, k):
            srcb, relb, dstb, _, normb, _, _, semi, _, _ = slots[b]
            be = (base_chunk + k) * c
            return (
                pltpu.make_async_copy(src_hbm.at[pl.ds(be, c)], srcb, semi),
                pltpu.make_async_copy(rel_hbm.at[pl.ds(be, c)], relb, semi),
                pltpu.make_async_copy(dst_hbm.at[pl.ds(be, c)], dstb, semi),
                pltpu.make_async_copy(norm_hbm.at[pl.ds(be, c)], normb, semi),
            )

        def issue_idx(b, k):
            for cp_ in idx_copies(b, k):
                cp_.start()

        def wait_idx(b, k):
            for cp_ in idx_copies(b, k):
                cp_.wait()

        def gidx_compute(b):
            srcb, relb, _, gidxb, _, _, _, _, _, _ = slots[b]
            for k16 in range(c // _LANES):
                sl = pl.ds(k16 * _LANES, _LANES)
                gidxb[sl] = relb[sl] * n_nodes + srcb[sl]

        def gather_copy(b):
            _, _, _, gidxb, _, rows, _, _, semg, _ = slots[b]
            return pltpu.make_async_copy(t_hbm.at[gidxb], rows, semg)

        def scale(b):
            _, _, dstb, _, normb, rows, dsts, _, _, _ = slots[b]

            @pl.loop(0, c, step=4)
            def _scale(i):
                for u in range(4):
                    ii = i + u
                    nb = plsc.load_gather(
                        normb, [jnp.full((_LANES,), ii, jnp.int32)])
                    for kk in range(nd16):
                        sl = pl.ds(kk * _LANES, _LANES)
                        rows[ii, sl] = rows[ii, sl] * nb

            for k16 in range(c // _LANES):
                sl = pl.ds(k16 * _LANES, _LANES)
                dsts[sl] = dstb[sl]

        def scatter_desc(b):
            _, _, _, _, _, rows, dsts, _, _, sema = slots[b]
            return pltpu.make_async_copy(rows, acc.at[dsts], sema)

        rows0 = slots[0][5]

        @pl.loop(0, c)
        def _zero_rows(i):
            for k in range(nd16):
                rows0[i, pl.ds(k * _LANES, _LANES)] = zero16

        row0 = sub * npt
        for jb in range(npt // c):
            pltpu.sync_copy(rows0, acc.at[pl.ds(row0 + jb * c, c)])

        plsc.subcore_barrier()

        for b in range(_N_SLOTS):
            issue_idx(b, b)
        for b in range(_N_SLOTS - 1):
            wait_idx(b, b)
            gidx_compute(b)
            gather_copy(b).start()

        def body(b, k, static_tail=False):
            b2 = (b + 2) % _N_SLOTS

            if not static_tail:
                @pl.when(k + 2 < cpt)
                def _prefetch_gather():
                    wait_idx(b2, k + 2)
                    gidx_compute(b2)

                    @pl.when(k >= 1)
                    def _drain_scatter():
                        scatter_desc(b2).wait()

                    gather_copy(b2).start()

            gather_copy(b).wait()
            scale(b)
            scatter_desc(b).start(add=True)

            if not static_tail:
                @pl.when(k + 3 < cpt)
                def _prefetch_idx():
                    issue_idx(b, k + 3)

        n_main = (cpt - 2) // _N_SLOTS

        @pl.loop(0, n_main)
        def _main(t):
            k = _N_SLOTS * t
            for b in range(_N_SLOTS):
                body(b, k + b)

        for k in range(_N_SLOTS * n_main, cpt):
            b = k % _N_SLOTS
            if k + 2 < cpt:
                b2 = (b + 2) % _N_SLOTS
                wait_idx(b2, k + 2)
                gidx_compute(b2)
                if k >= 1:
                    scatter_desc(b2).wait()
                gather_copy(b2).start()
            body(b, k, static_tail=True)

        for m in range(cpt - _N_SLOTS, cpt):
            scatter_desc(m % _N_SLOTS).wait()

        plsc.subcore_barrier()

        @pl.when(sub < full_tiles)
        def _write_full():
            pltpu.sync_copy(acc.at[pl.ds(row0, npt)],
                            out_hbm.at[pl.ds(core * n_nodes + row0, npt)])

        if tail_rows:
            @pl.when(sub == full_tiles)
            def _write_tail():
                pltpu.sync_copy(
                    acc.at[pl.ds(row0, tail_rows)],
                    out_hbm.at[pl.ds(core * n_nodes + row0, tail_rows)])

    return sck(t_flat, src, rel, dst, norm_flat)


def kernel(h, edge_index, rel_type, norm, W):
    n, d_in = h.shape
    r, _, d_out = W.shape
    e = rel_type.shape[0]
    transformed = _transform(h.astype(jnp.bfloat16),
                             W.astype(jnp.bfloat16)).reshape(r * n, d_out)
    src = edge_index[0]
    dst = edge_index[1]
    partial = _sc_edge_aggregate(transformed, src, rel_type, dst,
                                 norm.reshape(e), n)
    return _combine(partial.reshape(2, n, d_out))
